# baseline (speedup 1.0000x reference)
import jax
import jax.numpy as jnp
import numpy as np
from jax import lax
from jax.experimental import pallas as pl
from jax.experimental.pallas import tpu as pltpu
from jax.experimental.pallas import tpu_sc as plsc

S = 1024
NMAX = 64
D = 128
N_PAIRS = 63
SEQ_LEN = 188
FEAT = 2 * NMAX + D
EPS = 0.1
A = float(1.0 / np.sqrt(1.0 + EPS * EPS))
B = float(A * EPS / np.sqrt(D))

NC, NS = 2, 16
NW = NC * NS
LANES = 16
NVH = D // LANES
SB = S // NW
NSETS = 3
NSTEP = N_PAIRS // NSETS


def _sc_body(example_h, label_h, mus_label_h, mus_class_h, ne_h, no_h,
             out_h, et, lt, sets, isems, osems):
    wid = lax.axis_index("s") * NC + lax.axis_index("c")
    s0 = SB * wid

    pltpu.sync_copy(example_h.at[wid], et)
    pltpu.sync_copy(label_h.at[wid], lt)

    def in_copies(p, k):
        cn, ecl, lrf = sets[k]
        sem = isems[k]
        return (
            pltpu.make_async_copy(ne_h.at[p, pl.ds(s0, SB)], cn.at[0], sem),
            pltpu.make_async_copy(no_h.at[p, pl.ds(s0, SB)], cn.at[1], sem),
            pltpu.make_async_copy(
                mus_class_h.at[et.at[p, pl.ds(0, SB)]], ecl.at[0], sem),
            pltpu.make_async_copy(
                mus_class_h.at[et.at[p, pl.ds(SB, SB)]], ecl.at[1], sem),
            pltpu.make_async_copy(mus_label_h.at[lt.at[p]], lrf, sem),
        )

    def issue(copies):
        for c in copies:
            c.start()

    def drain(copies):
        for c in copies:
            c.wait()

    def add_noise(k):
        cn, ecl, _ = sets[k]

        def row(i, _):
            for v in range(NVH):
                sv = pl.ds(LANES * v, LANES)
                dv = pl.ds(D + LANES * v, LANES)
                ecl[0, i, dv] = ecl[0, i, dv] + B * cn[0, i, sv]
                ecl[1, i, dv] = ecl[1, i, dv] + B * cn[1, i, sv]
            return 0
        lax.fori_loop(0, SB, row, 0)

    for k in range(NSETS):
        issue(in_copies(k, k))

    def step(u, _):
        j = NSETS * u
        for k in range(NSETS):
            drain(in_copies(j + k, k))
            add_noise(k)
            start_out(j + k, k)
        for k in range(NSETS):
            @pl.when(j + NSETS + k < N_PAIRS)
            def _():
                drain(out_copies_full(j + k, k))
                issue(in_copies(j + NSETS + k, k))
        return 0

    def start_out(p, k):
        if k == NSETS - 1:
            @pl.when(p < N_PAIRS - 1)
            def _():
                issue(out_copies_full(p, k))

            @pl.when(p == N_PAIRS - 1)
            def _():
                issue(out_copies_last(p, k))
        else:
            issue(out_copies_full(p, k))

    def out_copies_full(p, k):
        _, ecl, lrf = sets[k]
        sem = osems[k]
        return (
            pltpu.make_async_copy(
                ecl, out_h.at[pl.ds(3 * p, 2), pl.ds(s0, SB)], sem),
            pltpu.make_async_copy(
                lrf, out_h.at[3 * p + 2, pl.ds(s0, SB)], sem),
        )

    def out_copies_last(p, k):
        _, ecl, _ = sets[k]
        sem = osems[k]
        return (
            pltpu.make_async_copy(
                ecl, out_h.at[pl.ds(3 * p, 2), pl.ds(s0, SB)], sem),
        )

    lax.fori_loop(0, NSTEP, step, 0)

    drain(out_copies_full(N_PAIRS - 3, 0))
    drain(out_copies_full(N_PAIRS - 2, 1))
    drain(out_copies_last(N_PAIRS - 1, 2))


def kernel(example, label, mus_label, mus_class, noise_even, noise_odd):
    mesh = plsc.VectorSubcoreMesh(core_axis_name="c", subcore_axis_name="s",
                                  num_cores=NC, num_subcores=NS)
    call = pl.kernel(
        _sc_body, mesh=mesh,
        out_type=jax.ShapeDtypeStruct((SEQ_LEN, S, FEAT), jnp.float32),
        scratch_types=[
            pltpu.VMEM((N_PAIRS, 2 * SB), jnp.int32),
            pltpu.VMEM((N_PAIRS, SB), jnp.int32),
            [[pltpu.VMEM((2, SB, D), jnp.float32),
              pltpu.VMEM((2, SB, FEAT), jnp.float32),
              pltpu.VMEM((SB, FEAT), jnp.float32)]
             for _ in range(NSETS)],
            [pltpu.SemaphoreType.DMA for _ in range(NSETS)],
            [pltpu.SemaphoreType.DMA for _ in range(NSETS)],
        ],
    )
    ne_t = jnp.transpose(noise_even, (1, 0, 2))
    no_t = jnp.transpose(noise_odd, (1, 0, 2))
    zpad = jnp.zeros((mus_class.shape[0], FEAT - D), jnp.float32)
    mc = jnp.concatenate([zpad, A * mus_class], axis=1)
    ml = jnp.concatenate([zpad, mus_label], axis=1)
    exw = jnp.transpose(example.astype(jnp.int32), (1, 0)) \
             .reshape(N_PAIRS, 2, NW, SB).transpose(2, 0, 1, 3) \
             .reshape(NW, N_PAIRS, 2 * SB)
    lbw = jnp.transpose(label.astype(jnp.int32), (1, 0)) \
             .reshape(N_PAIRS, NW, SB).transpose(1, 0, 2)
    out_t = call(exw, lbw, ml, mc, ne_t, no_t)
    return jnp.transpose(out_t, (1, 0, 2))


# speedup vs baseline: 1.1148x; 1.1148x over previous
import jax
import jax.numpy as jnp
import numpy as np
from jax import lax
from jax.experimental import pallas as pl
from jax.experimental.pallas import tpu as pltpu
from jax.experimental.pallas import tpu_sc as plsc

S = 1024
NMAX = 64
D = 128
N_PAIRS = 63
SEQ_LEN = 188
FEAT = 2 * NMAX + D
EPS = 0.1
A = float(1.0 / np.sqrt(1.0 + EPS * EPS))
B = float(A * EPS / np.sqrt(D))

NC, NS = 2, 16
NW = NC * NS
LANES = 16
NVH = D // LANES
SB = S // NW
NSETS = 3
NSTEP = N_PAIRS // NSETS


def _sc_body(example_h, label_h, mus_label_h, mus_class_h, ne_h, no_h,
             out_h, et, lt, sets, isems, osems):
    wid = lax.axis_index("s") * NC + lax.axis_index("c")
    s0 = SB * wid

    def zrow(r, _):
        for k in range(NSETS):
            _, ecl, lrf = sets[k]
            for v in range(NVH):
                sv = pl.ds(LANES * v, LANES)
                z = jnp.zeros((LANES,), jnp.float32)
                ecl[0, r, sv] = z
                ecl[1, r, sv] = z
                lrf[r, sv] = z
        return 0
    lax.fori_loop(0, SB, zrow, 0)

    pltpu.sync_copy(example_h.at[wid], et)
    pltpu.sync_copy(label_h.at[wid], lt)

    def in_copies(p, k):
        cn, ecl, lrf = sets[k]
        sem = isems[k]
        return (
            pltpu.make_async_copy(ne_h.at[p, pl.ds(s0, SB)], cn.at[0], sem),
            pltpu.make_async_copy(no_h.at[p, pl.ds(s0, SB)], cn.at[1], sem),
            pltpu.make_async_copy(
                mus_class_h.at[et.at[p, pl.ds(0, SB)]],
                ecl.at[0, :, pl.ds(D, D)], sem),
            pltpu.make_async_copy(
                mus_class_h.at[et.at[p, pl.ds(SB, SB)]],
                ecl.at[1, :, pl.ds(D, D)], sem),
            pltpu.make_async_copy(
                mus_label_h.at[lt.at[p]], lrf.at[:, pl.ds(D, D)], sem),
        )

    def issue(copies):
        for c in copies:
            c.start()

    def drain(copies):
        for c in copies:
            c.wait()

    def assemble(k):
        cn, ecl, _ = sets[k]

        def row(i, _):
            for v in range(NVH):
                sv = pl.ds(LANES * v, LANES)
                dv = pl.ds(D + LANES * v, LANES)
                ecl[0, i, dv] = ecl[0, i, dv] + B * cn[0, i, sv]
                ecl[1, i, dv] = ecl[1, i, dv] + B * cn[1, i, sv]
            return 0
        lax.fori_loop(0, SB, row, 0)

    for k in range(NSETS):
        issue(in_copies(k, k))

    def step(u, _):
        j = NSETS * u
        for k in range(NSETS):
            drain(in_copies(j + k, k))
            assemble(k)
            start_out(j + k, k)
        for k in range(NSETS):
            @pl.when(j + NSETS + k < N_PAIRS)
            def _():
                drain(out_copies_full(j + k, k))
                issue(in_copies(j + NSETS + k, k))
        return 0

    def start_out(p, k):
        if k == NSETS - 1:
            @pl.when(p < N_PAIRS - 1)
            def _():
                issue(out_copies_full(p, k))

            @pl.when(p == N_PAIRS - 1)
            def _():
                issue(out_copies_last(p, k))
        else:
            issue(out_copies_full(p, k))

    def out_copies_full(p, k):
        _, ecl, lrf = sets[k]
        sem = osems[k]
        return (
            pltpu.make_async_copy(
                ecl, out_h.at[pl.ds(3 * p, 2), pl.ds(s0, SB)], sem),
            pltpu.make_async_copy(
                lrf, out_h.at[3 * p + 2, pl.ds(s0, SB)], sem),
        )

    def out_copies_last(p, k):
        _, ecl, _ = sets[k]
        sem = osems[k]
        return (
            pltpu.make_async_copy(
                ecl, out_h.at[pl.ds(3 * p, 2), pl.ds(s0, SB)], sem),
        )

    lax.fori_loop(0, NSTEP, step, 0)

    drain(out_copies_full(N_PAIRS - 3, 0))
    drain(out_copies_full(N_PAIRS - 2, 1))
    drain(out_copies_last(N_PAIRS - 1, 2))


def kernel(example, label, mus_label, mus_class, noise_even, noise_odd):
    mesh = plsc.VectorSubcoreMesh(core_axis_name="c", subcore_axis_name="s",
                                  num_cores=NC, num_subcores=NS)
    call = pl.kernel(
        _sc_body, mesh=mesh,
        out_type=jax.ShapeDtypeStruct((SEQ_LEN, S, FEAT), jnp.float32),
        scratch_types=[
            pltpu.VMEM((N_PAIRS, 2 * SB), jnp.int32),
            pltpu.VMEM((N_PAIRS, SB), jnp.int32),
            [[pltpu.VMEM((2, SB, D), jnp.float32),
              pltpu.VMEM((2, SB, FEAT), jnp.float32),
              pltpu.VMEM((SB, FEAT), jnp.float32)]
             for _ in range(NSETS)],
            [pltpu.SemaphoreType.DMA for _ in range(NSETS)],
            [pltpu.SemaphoreType.DMA for _ in range(NSETS)],
        ],
    )
    ne_t = jnp.transpose(noise_even, (1, 0, 2))
    no_t = jnp.transpose(noise_odd, (1, 0, 2))
    mc = A * mus_class
    ml = mus_label
    exw = jnp.transpose(example.astype(jnp.int32), (1, 0)) \
             .reshape(N_PAIRS, 2, NW, SB).transpose(2, 0, 1, 3) \
             .reshape(NW, N_PAIRS, 2 * SB)
    lbw = jnp.transpose(label.astype(jnp.int32), (1, 0)) \
             .reshape(N_PAIRS, NW, SB).transpose(1, 0, 2)
    out_t = call(exw, lbw, ml, mc, ne_t, no_t)
    return jnp.transpose(out_t, (1, 0, 2))


# speedup vs baseline: 1.8714x; 1.6787x over previous
import jax
import jax.numpy as jnp
import numpy as np
from jax import lax
from jax.experimental import pallas as pl
from jax.experimental.pallas import tpu as pltpu
from jax.experimental.pallas import tpu_sc as plsc

S = 1024
NMAX = 64
D = 128
N_PAIRS = 63
N_EX = 2 * N_PAIRS
SEQ_LEN = 188
FEAT = 2 * NMAX + D
EPS = 0.1
A = float(1.0 / np.sqrt(1.0 + EPS * EPS))
B = float(A * EPS / np.sqrt(D))

NC, NS = 2, 16
NW = NC * NS
LANES = 16
NVH = D // LANES
SB = S // NW
NSETS = 3
NSTEP = N_PAIRS // NSETS


def _sc_body(example_h, label_h, mus_label_h, mus_class_h, ne_h, no_h,
             out_h, et, lt, zbuf, sets, isems, osems):
    wid = lax.axis_index("s") * NC + lax.axis_index("c")
    s0 = SB * wid

    def zrow(r, _):
        for v in range(NVH):
            zbuf[r // SB, r % SB, pl.ds(LANES * v, LANES)] = \
                jnp.zeros((LANES,), jnp.float32)
        return 0
    lax.fori_loop(0, 3 * SB, zrow, 0)

    pltpu.sync_copy(example_h.at[wid], et)
    pltpu.sync_copy(label_h.at[wid], lt)

    def in_copies(p, k):
        ceco, lr, ne, no = sets[k]
        sem = isems[k]
        return (
            pltpu.make_async_copy(
                mus_class_h.at[et.at[p, pl.ds(0, SB)]], ceco.at[0], sem),
            pltpu.make_async_copy(
                mus_class_h.at[et.at[p, pl.ds(SB, SB)]], ceco.at[1], sem),
            pltpu.make_async_copy(mus_label_h.at[lt.at[p]], lr, sem),
            pltpu.make_async_copy(ne_h.at[p, pl.ds(s0, SB)], ne, sem),
            pltpu.make_async_copy(no_h.at[p, pl.ds(s0, SB)], no, sem),
        )

    def issue(copies):
        for c in copies:
            c.start()

    def drain(copies):
        for c in copies:
            c.wait()

    def compute(k):
        ceco, _, ne, no = sets[k]

        def row(i, _):
            for v in range(NVH):
                sv = pl.ds(LANES * v, LANES)
                ceco[0, i, sv] = A * ceco[0, i, sv] + B * ne[i, sv]
                ceco[1, i, sv] = A * ceco[1, i, sv] + B * no[i, sv]
            return 0
        lax.fori_loop(0, SB, row, 0)

    for k in range(NSETS):
        issue(in_copies(k, k))

    def step(u, _):
        j = NSETS * u
        for k in range(NSETS):
            drain(in_copies(j + k, k))
            compute(k)
            start_out(j + k, k)
        for k in range(NSETS):
            @pl.when(j + NSETS + k < N_PAIRS)
            def _():
                drain(out_copies_full(j + k, k))
                issue(in_copies(j + NSETS + k, k))
        return 0

    def start_out(p, k):
        if k == NSETS - 1:
            @pl.when(p < N_PAIRS - 1)
            def _():
                issue(out_copies_full(p, k))

            @pl.when(p == N_PAIRS - 1)
            def _():
                issue(out_copies_last(p, k))
        else:
            issue(out_copies_full(p, k))

    def out_copies_full(p, k):
        ceco, lr, _, _ = sets[k]
        sem = osems[k]
        return (
            pltpu.make_async_copy(
                zbuf,
                out_h.at[pl.ds(3 * p, 3), pl.ds(s0, SB), pl.ds(0, D)], sem),
            pltpu.make_async_copy(
                ceco, out_h.at[pl.ds(3 * p, 2), pl.ds(s0, SB), pl.ds(D, D)], sem),
            pltpu.make_async_copy(
                lr, out_h.at[3 * p + 2, pl.ds(s0, SB), pl.ds(D, D)], sem),
        )

    def out_copies_last(p, k):
        ceco, _, _, _ = sets[k]
        sem = osems[k]
        return (
            pltpu.make_async_copy(
                zbuf.at[pl.ds(0, 2)],
                out_h.at[pl.ds(3 * p, 2), pl.ds(s0, SB), pl.ds(0, D)], sem),
            pltpu.make_async_copy(
                ceco, out_h.at[pl.ds(3 * p, 2), pl.ds(s0, SB), pl.ds(D, D)], sem),
        )

    lax.fori_loop(0, NSTEP, step, 0)

    drain(out_copies_full(N_PAIRS - 3, 0))
    drain(out_copies_full(N_PAIRS - 2, 1))
    drain(out_copies_last(N_PAIRS - 1, 2))


def kernel(example, label, mus_label, mus_class, noise_even, noise_odd):
    mesh = plsc.VectorSubcoreMesh(core_axis_name="c", subcore_axis_name="s",
                                  num_cores=NC, num_subcores=NS)
    call = pl.kernel(
        _sc_body, mesh=mesh,
        out_type=jax.ShapeDtypeStruct((SEQ_LEN, S, FEAT), jnp.float32),
        scratch_types=[
            pltpu.VMEM((N_PAIRS, 2 * SB), jnp.int32),
            pltpu.VMEM((N_PAIRS, SB), jnp.int32),
            pltpu.VMEM((3, SB, D), jnp.float32),
            [[pltpu.VMEM((2, SB, D), jnp.float32),
              pltpu.VMEM((SB, D), jnp.float32),
              pltpu.VMEM((SB, D), jnp.float32),
              pltpu.VMEM((SB, D), jnp.float32)]
             for _ in range(NSETS)],
            [pltpu.SemaphoreType.DMA for _ in range(NSETS)],
            [pltpu.SemaphoreType.DMA for _ in range(NSETS)],
        ],
    )
    ne_t = jnp.transpose(noise_even, (1, 0, 2))
    no_t = jnp.transpose(noise_odd, (1, 0, 2))
    exw = jnp.transpose(example.astype(jnp.int32), (1, 0)) \
             .reshape(N_PAIRS, 2, NW, SB).transpose(2, 0, 1, 3) \
             .reshape(NW, N_PAIRS, 2 * SB)
    lbw = jnp.transpose(label.astype(jnp.int32), (1, 0)) \
             .reshape(N_PAIRS, NW, SB).transpose(1, 0, 2)
    out_t = call(exw, lbw, mus_label, mus_class, ne_t, no_t)
    return jnp.transpose(out_t, (1, 0, 2))
